# baseline (speedup 1.0000x reference)
import functools

import jax
import jax.numpy as jnp
from jax import lax
from jax.experimental import pallas as pl
from jax.experimental.pallas import tpu as pltpu
from jax.experimental.pallas import tpu_sc as plsc

IN_SIZE = 1000000
OUT_SIZE = 64
BATCH = 16384
HIST = 50
TOTAL = BATCH * HIST

_info = plsc.get_sparse_core_info()
_NC, _NS = _info.num_cores, _info.num_subcores
NW = _NC * _NS
B_PER_W = TOTAL // NW
CHUNK = 640
NCHUNKS = B_PER_W // CHUNK


@functools.partial(
    pl.kernel,
    mesh=plsc.VectorSubcoreMesh(core_axis_name="c", subcore_axis_name="s"),
    out_type=jax.ShapeDtypeStruct((TOTAL, OUT_SIZE), jnp.float32),
    scratch_types=[
        pltpu.VMEM((B_PER_W,), jnp.int32),
        pltpu.VMEM((CHUNK, OUT_SIZE), jnp.float32),
        pltpu.SemaphoreType.DMA,
    ],
    compiler_params=pltpu.CompilerParams(use_tc_tiling_on_sc=False),
)
def _gather_kernel(table_hbm, idx_hbm, out_hbm, idx_v, rows_v, sem):
    wid = lax.axis_index("s") * _NC + lax.axis_index("c")
    base = wid * B_PER_W
    pltpu.sync_copy(idx_hbm.at[pl.ds(base, B_PER_W)], idx_v)

    def body(c, carry):
        off = pl.multiple_of(c * CHUNK, 8)
        pltpu.async_copy(
            table_hbm.at[idx_v.at[pl.ds(off, CHUNK)]], rows_v, sem
        ).wait()
        pltpu.sync_copy(rows_v, out_hbm.at[pl.ds(base + off, CHUNK)])
        return carry

    lax.fori_loop(0, NCHUNKS, body, 0)


def kernel(x, mat):
    flat_idx = x.reshape(-1)
    return _gather_kernel(mat, flat_idx)


# speedup vs baseline: 1.0184x; 1.0184x over previous
import functools

import jax
import jax.numpy as jnp
from jax import lax
from jax.experimental import pallas as pl
from jax.experimental.pallas import tpu as pltpu
from jax.experimental.pallas import tpu_sc as plsc

IN_SIZE = 1000000
OUT_SIZE = 64
BATCH = 16384
HIST = 50
TOTAL = BATCH * HIST

_info = plsc.get_sparse_core_info()
_NC, _NS = _info.num_cores, _info.num_subcores
NW = _NC * _NS
B_PER_W = TOTAL // NW
CHUNK = 640
NCHUNKS = B_PER_W // CHUNK
NPAIR = NCHUNKS // 2


@functools.partial(
    pl.kernel,
    mesh=plsc.VectorSubcoreMesh(core_axis_name="c", subcore_axis_name="s"),
    out_type=jax.ShapeDtypeStruct((TOTAL, OUT_SIZE), jnp.float32),
    scratch_types=[
        pltpu.VMEM((B_PER_W,), jnp.int32),
        pltpu.VMEM((CHUNK, OUT_SIZE), jnp.float32),
        pltpu.VMEM((CHUNK, OUT_SIZE), jnp.float32),
        pltpu.SemaphoreType.DMA,
        pltpu.SemaphoreType.DMA,
        pltpu.SemaphoreType.DMA,
        pltpu.SemaphoreType.DMA,
    ],
    compiler_params=pltpu.CompilerParams(use_tc_tiling_on_sc=False),
)
def _gather_kernel(
    table_hbm, idx_hbm, out_hbm, idx_v, rows_a, rows_b, ga, gb, wa, wb
):
    wid = lax.axis_index("s") * _NC + lax.axis_index("c")
    base = wid * B_PER_W
    rows = [rows_a, rows_b]
    gsem = [ga, gb]
    wsem = [wa, wb]

    pltpu.sync_copy(idx_hbm.at[pl.ds(base, B_PER_W)], idx_v)

    def g_copy(c, b):
        off = pl.multiple_of(c * CHUNK, 8)
        return pltpu.make_async_copy(
            table_hbm.at[idx_v.at[pl.ds(off, CHUNK)]], rows[b], gsem[b]
        )

    def w_copy(c, b):
        off = pl.multiple_of(c * CHUNK, 8)
        return pltpu.make_async_copy(
            rows[b], out_hbm.at[pl.ds(base + off, CHUNK)], wsem[b]
        )

    g_copy(0, 0).start()
    g_copy(1, 1).start()

    def body(i, carry):
        c = 2 * i
        g_copy(c, 0).wait()
        w_copy(c, 0).start()
        w_copy(c, 0).wait()
        g_copy(c + 2, 0).start()
        g_copy(c + 1, 1).wait()
        w_copy(c + 1, 1).start()
        w_copy(c + 1, 1).wait()
        g_copy(c + 3, 1).start()
        return carry

    lax.fori_loop(0, NPAIR - 1, body, 0)

    c = NCHUNKS - 2
    g_copy(c, 0).wait()
    w_copy(c, 0).start()
    g_copy(c + 1, 1).wait()
    w_copy(c + 1, 1).start()
    w_copy(c, 0).wait()
    w_copy(c + 1, 1).wait()


def kernel(x, mat):
    flat_idx = x.reshape(-1)
    return _gather_kernel(mat, flat_idx)
